# baseline (speedup 1.0000x reference)
import functools

import jax
import jax.numpy as jnp
from jax import lax
from jax.experimental import pallas as pl
from jax.experimental.pallas import tpu as pltpu
from jax.experimental.pallas import tpu_sc as plsc

N = 10000
E = 320000
D = 16
NSC = 2
NTILES = 16
CHUNK = 1000
NBUF = 5
K = 10
NPAD = 10112
RPT = NPAD // NTILES
NR = N * D // 128
NPR = NPAD * D // 128


def _make_agg(d=None, gather=True):
    if d is None:
        d = D
    mesh = plsc.VectorSubcoreMesh(core_axis_name="c", subcore_axis_name="s")
    nbuf = NBUF if gather else 1
    scratch = [
        pltpu.VMEM((K, CHUNK), jnp.int32),
        pltpu.VMEM((K, CHUNK), jnp.int32),
        pltpu.VMEM((nbuf, CHUNK, d), jnp.float32),
        pltpu.VMEM_SHARED((NPAD, d), jnp.float32),
    ] + [pltpu.SemaphoreType.DMA] * nbuf

    @functools.partial(
        pl.kernel,
        out_type=jax.ShapeDtypeStruct((NSC, NPAD, d), jnp.float32),
        mesh=mesh,
        scratch_types=scratch,
        compiler_params=pltpu.CompilerParams(use_tc_tiling_on_sc=False),
    )
    def agg(g_hbm, ei_hbm, zero_hbm, out_hbm, sidx, didx, rows, acc, *sems):
        c = lax.axis_index("c")
        s = lax.axis_index("s")
        pltpu.sync_copy(zero_hbm.at[pl.ds(s * RPT, RPT)],
                        acc.at[pl.ds(s * RPT, RPT)])
        if gather:
            pltpu.sync_copy(ei_hbm.at[0, c, s], sidx)
        pltpu.sync_copy(ei_hbm.at[1, c, s], didx)
        plsc.subcore_barrier()

        if True:
            if gather:
                for b in range(NBUF):
                    pltpu.async_copy(g_hbm.at[sidx.at[b]], rows.at[b], sems[b])

                def group(gi, carry):
                    for b in range(NBUF):
                        k = gi * NBUF + b
                        pltpu.make_async_copy(g_hbm.at[pl.ds(0, CHUNK)],
                                              rows.at[b], sems[b]).wait()
                        pltpu.sync_copy(rows.at[b], acc.at[didx.at[k]],
                                        add=True)
                        nk = k + NBUF

                        @pl.when(nk < K)
                        def _():
                            pltpu.async_copy(g_hbm.at[sidx.at[nk]],
                                             rows.at[b], sems[b])
                    return carry

                lax.fori_loop(0, K // NBUF, group, 0)
            else:
                pltpu.sync_copy(g_hbm, rows.at[0])

                def chunk(k, carry):
                    pltpu.sync_copy(rows.at[0], acc.at[didx.at[k]],
                                    add=True)
                    return carry

                lax.fori_loop(0, K, chunk, 0)

        plsc.subcore_barrier()
        pltpu.sync_copy(acc.at[pl.ds(s * RPT, RPT)],
                        out_hbm.at[c].at[pl.ds(s * RPT, RPT)])

    return agg


_agg = _make_agg()
_agg_ones8 = _make_agg(d=8, gather=False)


def _mm_body(x8_ref, kw_ref, u_ref):
    u_ref[...] = jnp.dot(x8_ref[...], kw_ref[...],
                         preferred_element_type=jnp.float32)


def _tc0_body(u_ref, dp_ref, pe_ref, po_ref, dinv_ref, g_ref):
    deg8 = dp_ref[0] + dp_ref[1] + 1.0
    dv8 = lax.rsqrt(deg8)
    a = jnp.dot(dv8, pe_ref[...], preferred_element_type=jnp.float32)
    b = jnp.dot(dv8, po_ref[...], preferred_element_type=jnp.float32)
    dinv = jnp.concatenate([a[:, None, :], b[:, None, :]], axis=1)
    dinv = dinv.reshape(NPR, 128)
    dinv_ref[...] = dinv
    g_ref[...] = dinv[:NR] * u_ref[...]


def _mid_body(dinv_ref, sp_ref, g_ref, bt_ref, kw_ref, o_ref):
    dv = dinv_ref[...][:NR]
    s = sp_ref[0][:NR] + sp_ref[1][:NR] + g_ref[...]
    h = dv * s + bt_ref[...]
    a = dv * (h * jnp.tanh(jax.nn.softplus(h)))
    o_ref[...] = jnp.dot(a, kw_ref[...], preferred_element_type=jnp.float32)


def _fin_body(dinv_ref, sp_ref, g_ref, bt_ref, pswap_ref, o_ref):
    dv = dinv_ref[...][:NR]
    t = dv * (sp_ref[0][:NR] + sp_ref[1][:NR] + g_ref[...]) + bt_ref[...]
    u = jnp.dot(t, pswap_ref[...], preferred_element_type=jnp.float32)
    m = jnp.maximum(t, u)
    o_ref[...] = t - m - jnp.log(jnp.exp(t - m) + jnp.exp(u - m))


def _tc_mm(x8, kw):
    return pl.pallas_call(
        _mm_body,
        out_shape=jax.ShapeDtypeStruct((NR, 128), jnp.float32),
    )(x8, kw)


def _tc0(u, dp, pe, po):
    return pl.pallas_call(
        _tc0_body,
        out_shape=(jax.ShapeDtypeStruct((NPR, 128), jnp.float32),
                   jax.ShapeDtypeStruct((NR, 128), jnp.float32)),
    )(u, dp, pe, po)


def _tc_mid(dinv, sp, g, bt, kw):
    return pl.pallas_call(
        _mid_body,
        out_shape=jax.ShapeDtypeStruct((NR, 128), jnp.float32),
    )(dinv, sp, g, bt, kw)


def _tc_fin(dinv, sp, g, bt, pswap):
    return pl.pallas_call(
        _fin_body,
        out_shape=jax.ShapeDtypeStruct((NR, 128), jnp.float32),
    )(dinv, sp, g, bt, pswap)


def _packw(w):
    return jnp.kron(jnp.eye(8, dtype=jnp.float32), w)


def _packb(b):
    return jnp.tile(b, 8).reshape(1, 128)


def kernel(x, edge_index, W1, b1, W2, b2, W3, b3, W4, b4):
    ei_r = edge_index.reshape(2, NSC, NTILES, K, CHUNK)
    zero = jnp.zeros((NPAD, D), jnp.float32)
    zero8 = jnp.zeros((NPAD, 8), jnp.float32)
    ones8 = jnp.ones((CHUNK, 8), jnp.float32)
    sel = jnp.zeros((8, 128), jnp.float32).at[
        jnp.repeat(jnp.arange(8), 16),
        jnp.arange(128)].set(1.0)
    lane8 = jax.nn.one_hot(jnp.arange(8) * 8, 128, dtype=jnp.float32)
    pe = lane8.T @ sel
    lane8o = jax.nn.one_hot(64 + jnp.arange(8) * 8, 128, dtype=jnp.float32)
    po = lane8o.T @ sel

    x8 = x.reshape(NR, 1024)
    kw1 = jnp.kron(jnp.eye(8, dtype=jnp.float32),
                   jnp.pad(W1, ((0, 0), (0, 8))))
    kw2 = _packw(jnp.pad(W2, ((0, 8), (0, 0))))
    kw3 = _packw(jnp.pad(W3, ((0, 0), (0, 8))))
    kw4 = _packw(jnp.pad(W4, ((0, 8), (0, 14))))
    bt1 = _packb(jnp.pad(b1, (0, 8)))
    bt2 = _packb(b2)
    bt3 = _packb(jnp.pad(b3, (0, 8)))
    bt4 = _packb(jnp.pad(b4, (0, 14)))
    pswap = jnp.kron(jnp.eye(64, dtype=jnp.float32),
                     jnp.array([[0.0, 1.0], [1.0, 0.0]], jnp.float32))

    degp = _agg_ones8(ones8, ei_r, zero8)
    u1 = _tc_mm(x8, kw1)
    dinv, g1 = _tc0(u1, degp.reshape(NSC, NPAD * 8 // 128, 128), pe, po)

    sp = _agg(g1.reshape(N, D), ei_r, zero)
    g2 = _tc_mid(dinv, sp.reshape(NSC, NPR, 128), g1, bt1, kw2)

    sp = _agg(g2.reshape(N, D), ei_r, zero)
    g3 = _tc_mid(dinv, sp.reshape(NSC, NPR, 128), g2, bt2, kw3)

    sp = _agg(g3.reshape(N, D), ei_r, zero)
    g4 = _tc_mid(dinv, sp.reshape(NSC, NPR, 128), g3, bt3, kw4)

    sp = _agg(g4.reshape(N, D), ei_r, zero)
    ls = _tc_fin(dinv, sp.reshape(NSC, NPR, 128), g4, bt4, pswap)
    return ls.reshape(N, D)[:, :2]


# speedup vs baseline: 1.0129x; 1.0129x over previous
import functools

import jax
import jax.numpy as jnp
from jax import lax
from jax.experimental import pallas as pl
from jax.experimental.pallas import tpu as pltpu
from jax.experimental.pallas import tpu_sc as plsc

N = 10000
E = 320000
D = 16
NSC = 2
NTILES = 16
CHUNK = 1000
NBUF = 5
K = 10
NPAD = 10112
RPT = NPAD // NTILES
NR = N * D // 128
NPR = NPAD * D // 128


def _make_agg(d=None, gather=True):
    if d is None:
        d = D
    mesh = plsc.VectorSubcoreMesh(core_axis_name="c", subcore_axis_name="s")
    nbuf = NBUF if gather else 1
    scratch = [
        pltpu.VMEM((K, CHUNK), jnp.int32),
        pltpu.VMEM((K, CHUNK), jnp.int32),
        pltpu.VMEM((nbuf, CHUNK, d), jnp.float32),
        pltpu.VMEM_SHARED((NPAD, d), jnp.float32),
    ] + [pltpu.SemaphoreType.DMA] * nbuf

    @functools.partial(
        pl.kernel,
        out_type=jax.ShapeDtypeStruct((NSC, NPAD, d), jnp.float32),
        mesh=mesh,
        scratch_types=scratch,
        compiler_params=pltpu.CompilerParams(use_tc_tiling_on_sc=False),
    )
    def agg(g_hbm, ei_hbm, zero_hbm, out_hbm, sidx, didx, rows, acc, *sems):
        c = lax.axis_index("c")
        s = lax.axis_index("s")
        pltpu.sync_copy(zero_hbm.at[pl.ds(s * RPT, RPT)],
                        acc.at[pl.ds(s * RPT, RPT)])
        if gather:
            pltpu.sync_copy(ei_hbm.at[0, c, s], sidx)
        pltpu.sync_copy(ei_hbm.at[1, c, s], didx)
        plsc.subcore_barrier()

        if True:
            if gather:
                for b in range(NBUF):
                    pltpu.async_copy(g_hbm.at[sidx.at[b]], rows.at[b], sems[b])

                def group(gi, carry):
                    for b in range(NBUF):
                        k = gi * NBUF + b
                        pltpu.make_async_copy(g_hbm.at[pl.ds(0, CHUNK)],
                                              rows.at[b], sems[b]).wait()
                        pltpu.sync_copy(rows.at[b], acc.at[didx.at[k]],
                                        add=True)
                        nk = k + NBUF

                        @pl.when(nk < K)
                        def _():
                            pltpu.async_copy(g_hbm.at[sidx.at[nk]],
                                             rows.at[b], sems[b])
                    return carry

                lax.fori_loop(0, K // NBUF, group, 0)
            else:
                pltpu.sync_copy(g_hbm, rows.at[0])

                def chunk(k, carry):
                    pltpu.sync_copy(rows.at[0], acc.at[didx.at[k]],
                                    add=True)
                    return carry

                lax.fori_loop(0, K, chunk, 0)

        plsc.subcore_barrier()
        pltpu.sync_copy(acc.at[pl.ds(s * RPT, RPT)],
                        out_hbm.at[c].at[pl.ds(s * RPT, RPT)])

    return agg


_agg = _make_agg()
_agg_ones = _make_agg(gather=False)


def _mm_body(x8_ref, kw_ref, u_ref):
    u_ref[...] = jnp.dot(x8_ref[...], kw_ref[...],
                         preferred_element_type=jnp.float32)


def _tc0_body(u_ref, dp_ref, dinv_ref, g_ref):
    deg = dp_ref[0] + dp_ref[1] + 1.0
    dinv = lax.rsqrt(deg)
    dinv_ref[...] = dinv
    g_ref[...] = dinv[:NR] * u_ref[...]


def _mid_body(dinv_ref, sp_ref, g_ref, bt_ref, kw_ref, o_ref):
    dv = dinv_ref[...][:NR]
    s = sp_ref[0][:NR] + sp_ref[1][:NR] + g_ref[...]
    h = dv * s + bt_ref[...]
    a = dv * (h * jnp.tanh(jax.nn.softplus(h)))
    o_ref[...] = jnp.dot(a, kw_ref[...], preferred_element_type=jnp.float32)


def _fin_body(dinv_ref, sp_ref, g_ref, bt_ref, pswap_ref, o_ref):
    dv = dinv_ref[...][:NR]
    t = dv * (sp_ref[0][:NR] + sp_ref[1][:NR] + g_ref[...]) + bt_ref[...]
    u = jnp.dot(t, pswap_ref[...], preferred_element_type=jnp.float32)
    m = jnp.maximum(t, u)
    o_ref[...] = t - m - jnp.log(jnp.exp(t - m) + jnp.exp(u - m))


def _tc_mm(x8, kw):
    return pl.pallas_call(
        _mm_body,
        out_shape=jax.ShapeDtypeStruct((NR, 128), jnp.float32),
    )(x8, kw)


def _tc0(u, dp):
    return pl.pallas_call(
        _tc0_body,
        out_shape=(jax.ShapeDtypeStruct((NPR, 128), jnp.float32),
                   jax.ShapeDtypeStruct((NR, 128), jnp.float32)),
    )(u, dp)


def _tc_mid(dinv, sp, g, bt, kw):
    return pl.pallas_call(
        _mid_body,
        out_shape=jax.ShapeDtypeStruct((NR, 128), jnp.float32),
    )(dinv, sp, g, bt, kw)


def _tc_fin(dinv, sp, g, bt, pswap):
    return pl.pallas_call(
        _fin_body,
        out_shape=jax.ShapeDtypeStruct((NR, 128), jnp.float32),
    )(dinv, sp, g, bt, pswap)


def _packw(w):
    return jnp.kron(jnp.eye(8, dtype=jnp.float32), w)


def _packb(b):
    return jnp.tile(b, 8).reshape(1, 128)


def kernel(x, edge_index, W1, b1, W2, b2, W3, b3, W4, b4):
    ei_r = edge_index.reshape(2, NSC, NTILES, K, CHUNK)
    zero = jnp.zeros((NPAD, D), jnp.float32)
    ones = jnp.ones((CHUNK, D), jnp.float32)

    x8 = x.reshape(NR, 1024)
    kw1 = jnp.kron(jnp.eye(8, dtype=jnp.float32),
                   jnp.pad(W1, ((0, 0), (0, 8))))
    kw2 = _packw(jnp.pad(W2, ((0, 8), (0, 0))))
    kw3 = _packw(jnp.pad(W3, ((0, 0), (0, 8))))
    kw4 = _packw(jnp.pad(W4, ((0, 8), (0, 14))))
    bt1 = _packb(jnp.pad(b1, (0, 8)))
    bt2 = _packb(b2)
    bt3 = _packb(jnp.pad(b3, (0, 8)))
    bt4 = _packb(jnp.pad(b4, (0, 14)))
    pswap = jnp.kron(jnp.eye(64, dtype=jnp.float32),
                     jnp.array([[0.0, 1.0], [1.0, 0.0]], jnp.float32))

    degp = _agg_ones(ones, ei_r, zero)
    u1 = _tc_mm(x8, kw1)
    dinv, g1 = _tc0(u1, degp.reshape(NSC, NPR, 128))

    sp = _agg(g1.reshape(N, D), ei_r, zero)
    g2 = _tc_mid(dinv, sp.reshape(NSC, NPR, 128), g1, bt1, kw2)

    sp = _agg(g2.reshape(N, D), ei_r, zero)
    g3 = _tc_mid(dinv, sp.reshape(NSC, NPR, 128), g2, bt2, kw3)

    sp = _agg(g3.reshape(N, D), ei_r, zero)
    g4 = _tc_mid(dinv, sp.reshape(NSC, NPR, 128), g3, bt3, kw4)

    sp = _agg(g4.reshape(N, D), ei_r, zero)
    ls = _tc_fin(dinv, sp.reshape(NSC, NPR, 128), g4, bt4, pswap)
    return ls.reshape(N, D)[:, :2]
